# Initial kernel scaffold; baseline (speedup 1.0000x reference)
import jax
import jax.numpy as jnp
from jax.experimental import pallas as pl


def kernel(img, img_mask, img_revert_idx, txt, txt_mask, txt_revert_idx, mask_token, pos_enc_2d, pe_nlp):
    raise NotImplementedError("write your pallas kernel here")


# speedup vs baseline: 3.4515x; 3.4515x over previous
import functools

import jax
import jax.numpy as jnp
from jax import lax
from jax.experimental import pallas as pl
from jax.experimental.pallas import tpu as pltpu
from jax.experimental.pallas import tpu_sc as plsc

L = 16


def _build(B, S_img, N_img, S_txt, N_txt, D):
    KD = D // L
    NW = 32
    mesh = plsc.VectorSubcoreMesh(core_axis_name="c", subcore_axis_name="s")

    def body(imgf, imaskf, iidxT, ipos, txtf, tmaskf, tidxT, tpos, mt,
             img_out, txt_out,
             mt_v, maski_v, maskt_v, idx_v, gidx_v, valf_v, pos_v, pmt_v,
             rows_v, sem):
        wid = lax.axis_index("s") * 2 + lax.axis_index("c")

        pltpu.sync_copy(mt, mt_v)
        pltpu.sync_copy(imaskf, maski_v)
        pltpu.sync_copy(tmaskf, maskt_v)

        def process(n, S, srcf, mask_v, idxT, posT, out):
            pltpu.sync_copy(idxT.at[n], idx_v)
            pltpu.sync_copy(posT.at[n], pos_v)
            for j in range(B // L):
                sl = pl.ds(j * L, L)
                srcv = idx_v[sl]
                srcc = jnp.minimum(srcv, S - 1)
                bvec = lax.iota(jnp.int32, L) + (j * L)
                gidx = bvec * S + srcc
                gidx_v[sl] = gidx
                mval = plsc.load_gather(mask_v, [gidx])
                valid = (srcv < S) & (mval == 1)
                valf_v[sl] = valid.astype(jnp.float32)
            pltpu.async_copy(srcf.at[gidx_v], rows_v, sem).wait()
            for k in range(KD):
                s = pl.ds(k * L, L)
                pmt_v[s] = pos_v[s] + mt_v[s]

            def bbody(b, carry):
                f = valf_v[pl.ds(b, L)][0]
                for k in range(KD):
                    s = pl.ds(k * L, L)
                    r = rows_v[b, s]
                    rows_v[b, s] = pmt_v[s] + f * (r - mt_v[s])
                return carry

            lax.fori_loop(0, B, bbody, 0)
            pltpu.sync_copy(rows_v, out.at[:, n, :])

        def img_body(i, carry):
            n = wid + NW * i

            @pl.when(n < N_img)
            def _():
                process(n, S_img, imgf, maski_v, iidxT, ipos, img_out)

            return carry

        lax.fori_loop(0, (N_img + NW - 1) // NW, img_body, 0)

        def txt_body(i, carry):
            n = wid + NW * i
            process(n, S_txt, txtf, maskt_v, tidxT, tpos, txt_out)
            return carry

        lax.fori_loop(0, N_txt // NW, txt_body, 0)

    return pl.kernel(
        body,
        mesh=mesh,
        compiler_params=pltpu.CompilerParams(needs_layout_passes=False),
        out_type=(
            jax.ShapeDtypeStruct((B, N_img, D), jnp.float32),
            jax.ShapeDtypeStruct((B, N_txt, D), jnp.float32),
        ),
        scratch_types=[
            pltpu.VMEM((D,), jnp.float32),
            pltpu.VMEM((B * S_img,), jnp.int32),
            pltpu.VMEM((B * S_txt,), jnp.int32),
            pltpu.VMEM((B,), jnp.int32),
            pltpu.VMEM((B,), jnp.int32),
            pltpu.VMEM((B + L,), jnp.float32),
            pltpu.VMEM((D,), jnp.float32),
            pltpu.VMEM((D,), jnp.float32),
            pltpu.VMEM((B, D), jnp.float32),
            pltpu.SemaphoreType.DMA,
        ],
    )


@jax.jit
def kernel(img, img_mask, img_revert_idx, txt, txt_mask, txt_revert_idx,
           mask_token, pos_enc_2d, pe_nlp):
    B, S_img, D = img.shape
    N_img = img_revert_idx.shape[1]
    S_txt = txt.shape[1]
    N_txt = txt_revert_idx.shape[1]

    fn = _build(B, S_img, N_img, S_txt, N_txt, D)
    img_out, txt_out = fn(
        img.reshape(B * S_img, D),
        img_mask.reshape(-1),
        img_revert_idx.T,
        pos_enc_2d,
        txt.reshape(B * S_txt, D),
        txt_mask.reshape(-1),
        txt_revert_idx.T,
        pe_nlp[:N_txt],
        mask_token.reshape(D),
    )
    return (img_out, txt_out)


# speedup vs baseline: 7.2520x; 2.1011x over previous
import jax
import jax.numpy as jnp
from jax import lax
from jax.experimental import pallas as pl
from jax.experimental.pallas import tpu as pltpu
from jax.experimental.pallas import tpu_sc as plsc

L = 16
HB = 32


def _build(B, S_img, N_img, S_txt, N_txt, D):
    KD = D // L
    NW = 32
    CNT_MAX = (N_img + NW - 1) // NW
    mesh = plsc.VectorSubcoreMesh(core_axis_name="c", subcore_axis_name="s")

    def body(imgf, imaskf, iidxT, ipos, txtf, tmaskf, tidxT, tpos, mt,
             img_out, txt_out,
             mt_v, maski_v, maskt_v, idx_all, pos_all, pmt_all,
             gidx_v, valf_v, rows_v, outb_v,
             hsem, psem, gsem, ssem):
        wid = lax.axis_index("s") * 2 + lax.axis_index("c")

        pltpu.sync_copy(mt, mt_v)
        pltpu.sync_copy(imaskf, maski_v)
        pltpu.sync_copy(tmaskf, maskt_v)

        def phase(S, cnt, srcf, mask_v, idxT, posT, out):
            def hdr_issue(j, c):
                n = wid + NW * j
                pltpu.make_async_copy(idxT.at[n], idx_all.at[j], hsem).start()
                pltpu.make_async_copy(posT.at[n], pos_all.at[j], psem).start()
                return c

            lax.fori_loop(0, cnt, hdr_issue, 0)

            def hdr_drain(j, c):
                pltpu.make_async_copy(idxT.at[wid], idx_all.at[0], hsem).wait()
                pltpu.make_async_copy(posT.at[wid], pos_all.at[0], psem).wait()
                return c

            lax.fori_loop(0, cnt, hdr_drain, 0)

            def pmt_body(j, c):
                for k in range(KD):
                    s = pl.ds(k * L, L)
                    pmt_all[j, s] = pos_all[j, s] + mt_v[s]
                return c

            lax.fori_loop(0, cnt, pmt_body, 0)

            def compute_idx(h):
                slot = lax.rem(h, 2)
                j = lax.div(h, 2)
                boff = lax.rem(h, 2) * HB
                for g in range(HB // L):
                    sl = pl.ds(g * L, L)
                    srcv = idx_all[j, pl.ds(boff + g * L, L)]
                    srcc = jnp.minimum(srcv, S - 1)
                    bvec = lax.iota(jnp.int32, L) + (g * L) + boff
                    gidx = bvec * S + srcc
                    gidx_v[slot, sl] = gidx
                    mval = plsc.load_gather(mask_v, [gidx])
                    valid = (srcv < S) & (mval == 1)
                    valf_v[slot, sl] = valid.astype(jnp.float32)

            def mk_gather(h):
                slot = lax.rem(h, 2)
                return pltpu.make_async_copy(
                    srcf.at[gidx_v.at[slot]], rows_v.at[slot], gsem.at[slot])

            def mk_store(h):
                slot = lax.rem(h, 2)
                j = lax.div(h, 2)
                n = wid + NW * j
                boff = lax.rem(h, 2) * HB
                return pltpu.make_async_copy(
                    outb_v.at[slot], out.at[pl.ds(boff, HB), n, :],
                    ssem.at[slot])

            def blend(h):
                slot = lax.rem(h, 2)
                j = lax.div(h, 2)
                for g in range(HB // L):
                    fv = valf_v[slot, pl.ds(g * L, L)]
                    fsp = [jnp.full((L,), fv[i], jnp.float32)
                           for i in range(L)]

                    def kb(k, c):
                        ks = pl.ds(k * L, L)
                        pmtk = pmt_all[j, ks]
                        mtk = mt_v[ks]
                        for i in range(L):
                            b = g * L + i
                            r = rows_v[slot, b, ks]
                            outb_v[slot, b, ks] = pmtk + fsp[i] * (r - mtk)
                        return c

                    lax.fori_loop(0, KD, kb, 0)

            H = 2 * cnt
            compute_idx(0)
            mk_gather(0).start()

            def loop(h, c):
                @pl.when(h + 1 < H)
                def _():
                    compute_idx(h + 1)
                    mk_gather(h + 1).start()

                mk_gather(h).wait()

                @pl.when(h >= 2)
                def _():
                    mk_store(h - 2).wait()

                blend(h)
                mk_store(h).start()
                return c

            lax.fori_loop(0, H, loop, 0)
            mk_store(H - 2).wait()
            mk_store(H - 1).wait()

        cnt_img = (N_img - wid + NW - 1) // NW
        phase(S_img, cnt_img, imgf, maski_v, iidxT, ipos, img_out)
        phase(S_txt, N_txt // NW, txtf, maskt_v, tidxT, tpos, txt_out)

    return pl.kernel(
        body,
        mesh=mesh,
        compiler_params=pltpu.CompilerParams(needs_layout_passes=False),
        out_type=(
            jax.ShapeDtypeStruct((B, N_img, D), jnp.float32),
            jax.ShapeDtypeStruct((B, N_txt, D), jnp.float32),
        ),
        scratch_types=[
            pltpu.VMEM((D,), jnp.float32),
            pltpu.VMEM((B * S_img,), jnp.int32),
            pltpu.VMEM((B * S_txt,), jnp.int32),
            pltpu.VMEM((CNT_MAX, B), jnp.int32),
            pltpu.VMEM((CNT_MAX, D), jnp.float32),
            pltpu.VMEM((CNT_MAX, D), jnp.float32),
            pltpu.VMEM((2, HB), jnp.int32),
            pltpu.VMEM((2, HB), jnp.float32),
            pltpu.VMEM((2, HB, D), jnp.float32),
            pltpu.VMEM((2, HB, D), jnp.float32),
            pltpu.SemaphoreType.DMA,
            pltpu.SemaphoreType.DMA,
            pltpu.SemaphoreType.DMA((2,)),
            pltpu.SemaphoreType.DMA((2,)),
        ],
    )


@jax.jit
def kernel(img, img_mask, img_revert_idx, txt, txt_mask, txt_revert_idx,
           mask_token, pos_enc_2d, pe_nlp):
    B, S_img, D = img.shape
    N_img = img_revert_idx.shape[1]
    S_txt = txt.shape[1]
    N_txt = txt_revert_idx.shape[1]

    fn = _build(B, S_img, N_img, S_txt, N_txt, D)
    img_out, txt_out = fn(
        img.reshape(B * S_img, D),
        img_mask.reshape(-1),
        img_revert_idx.T,
        pos_enc_2d,
        txt.reshape(B * S_txt, D),
        txt_mask.reshape(-1),
        txt_revert_idx.T,
        pe_nlp[:N_txt],
        mask_token.reshape(D),
    )
    return (img_out, txt_out)
